# baseline (speedup 1.0000x reference)
import functools

import jax
import jax.numpy as jnp
from jax import lax
from jax.experimental import pallas as pl
from jax.experimental.pallas import tpu as pltpu
from jax.experimental.pallas import tpu_sc as plsc

N = 10000
H = 128
E = 320000
PAD = 16
W = H + PAD

NC = 2
NS = 16
NPAD = N
RPT = NPAD // NS
EPC = E // NC
EPT = EPC // NS
C = 40
VPR = W // 16
G = 50
NCHUNK = EPT // C
NGROUP = NCHUNK // G


def _prologue_body(x_ref, ws_ref, wt_ref, b1_ref, ps_ref, pt_ref):
    x = x_ref[...]
    ps = jnp.dot(x, ws_ref[...], preferred_element_type=jnp.float32)
    pt = jnp.dot(x, wt_ref[...], preferred_element_type=jnp.float32) + b1_ref[...]
    col = lax.broadcasted_iota(jnp.int32, (x.shape[0], PAD), 1)
    pad = jnp.where(col == 0, 0.5, 0.0).astype(jnp.float32)
    ps_ref[...] = jnp.concatenate([ps, pad], axis=1)
    pt_ref[...] = jnp.concatenate([pt, pad], axis=1)


def _sc_edge_body(ps_hbm, pt_hbm, edge_hbm, out_hbm,
                  sidx, tidx, rs0, rt0, rs1, rt1, w0, w1, acc,
                  sem_s0, sem_t0, sem_s1, sem_t1, sem_w0, sem_w1):
    cid = lax.axis_index("c")
    sid = lax.axis_index("s")

    @pl.loop(0, C)
    def _zero_rows(i):
        for j in range(VPR):
            w0[i, pl.ds(j * 16, 16)] = jnp.zeros((16,), jnp.float32)

    row0 = sid * RPT
    for t in range(15):
        pltpu.sync_copy(w0, acc.at[pl.ds(row0 + t * C, C)])
    pltpu.sync_copy(w0.at[pl.ds(0, 25)], acc.at[pl.ds(row0 + 600, 25)])
    plsc.subcore_barrier()

    rbase = (cid * NS + sid) * NCHUNK

    def start_g(k, rs, rt, sem_s, sem_t):
        pltpu.async_copy(ps_hbm.at[sidx.at[k]], rs, sem_s)
        pltpu.async_copy(pt_hbm.at[tidx.at[k]], rt, sem_t)

    def compute(k, rs, rt, w, sem_s, sem_t):
        pltpu.make_async_copy(ps_hbm.at[sidx.at[k]], rs, sem_s).wait()
        pltpu.make_async_copy(pt_hbm.at[tidx.at[k]], rt, sem_t).wait()

        @pl.loop(0, C, unroll=4)
        def _relu_rows(i):
            for j in range(VPR):
                sl = pl.ds(j * 16, 16)
                w[i, sl] = jnp.maximum(rs[i, sl] + rt[i, sl], 0.0)

    def start_sc(k, w, sem_w):
        pltpu.async_copy(w, acc.at[tidx.at[k]], sem_w, add=True)

    def wait_sc(k, w, sem_w):
        pltpu.make_async_copy(w, acc.at[tidx.at[k]], sem_w).wait()

    @pl.loop(0, NGROUP)
    def _group(g):
        grow = rbase + g * G
        pltpu.sync_copy(edge_hbm.at[0, pl.ds(grow, G)], sidx)
        pltpu.sync_copy(edge_hbm.at[1, pl.ds(grow, G)], tidx)

        start_g(0, rs0, rt0, sem_s0, sem_t0)
        start_g(1, rs1, rt1, sem_s1, sem_t1)
        compute(0, rs0, rt0, w0, sem_s0, sem_t0)
        start_sc(0, w0, sem_w0)
        start_g(2, rs0, rt0, sem_s0, sem_t0)
        compute(1, rs1, rt1, w1, sem_s1, sem_t1)
        start_sc(1, w1, sem_w1)
        start_g(3, rs1, rt1, sem_s1, sem_t1)

        @pl.loop(0, (G - 4) // 2)
        def _pair(m):
            k = 2 * m + 2
            compute(k, rs0, rt0, w0, sem_s0, sem_t0)
            wait_sc(k - 2, w0, sem_w0)
            start_sc(k, w0, sem_w0)
            start_g(k + 2, rs0, rt0, sem_s0, sem_t0)
            compute(k + 1, rs1, rt1, w1, sem_s1, sem_t1)
            wait_sc(k - 1, w1, sem_w1)
            start_sc(k + 1, w1, sem_w1)
            start_g(k + 3, rs1, rt1, sem_s1, sem_t1)

        compute(G - 2, rs0, rt0, w0, sem_s0, sem_t0)
        wait_sc(G - 4, w0, sem_w0)
        start_sc(G - 2, w0, sem_w0)
        compute(G - 1, rs1, rt1, w1, sem_s1, sem_t1)
        wait_sc(G - 3, w1, sem_w1)
        start_sc(G - 1, w1, sem_w1)
        wait_sc(G - 2, w0, sem_w0)
        wait_sc(G - 1, w1, sem_w1)

    plsc.subcore_barrier()
    pltpu.sync_copy(acc.at[pl.ds(row0, RPT)], out_hbm.at[cid, pl.ds(row0, RPT)])


def _epilogue_body(parts_ref, x_ref, w2m_ref, b2m_ref, wus_ref, wua_ref,
                   b1u_ref, w2u_ref, b2u_ref, g_ref, bt_ref, o_ref):
    p = parts_ref[0] + parts_ref[1]
    aggh = p[:, :H]
    cnt = p[:, H:H + 1]
    mh = aggh / jnp.maximum(cnt, 1.0)
    agg = (jnp.dot(mh, w2m_ref[...], preferred_element_type=jnp.float32)
           + b2m_ref[...] * (cnt > 0).astype(jnp.float32))
    x = x_ref[...]
    h2 = jnp.maximum(
        jnp.dot(x, wus_ref[...], preferred_element_type=jnp.float32)
        + jnp.dot(agg, wua_ref[...], preferred_element_type=jnp.float32)
        + b1u_ref[...], 0.0)
    upd = jnp.dot(h2, w2u_ref[...], preferred_element_type=jnp.float32) + b2u_ref[...]
    y = x + upd
    mu = jnp.mean(y, axis=1, keepdims=True)
    var = jnp.mean((y - mu) * (y - mu), axis=1, keepdims=True)
    o_ref[...] = (y - mu) * lax.rsqrt(var + 1e-5) * g_ref[...] + bt_ref[...]


def kernel(node_features, edge_index, W1m, b1m, W2m, b2m,
           W1u, b1u, W2u, b2u, gamma, beta):
    flat = node_features.reshape(N, H)

    RP = 1000
    ps, pt = pl.pallas_call(
        _prologue_body,
        grid=(N // RP,),
        in_specs=[
            pl.BlockSpec((RP, H), lambda i: (i, 0)),
            pl.BlockSpec((H, H), lambda i: (0, 0)),
            pl.BlockSpec((H, H), lambda i: (0, 0)),
            pl.BlockSpec((1, H), lambda i: (0, 0)),
        ],
        out_specs=[
            pl.BlockSpec((RP, W), lambda i: (i, 0)),
            pl.BlockSpec((RP, W), lambda i: (i, 0)),
        ],
        out_shape=[
            jax.ShapeDtypeStruct((N, W), jnp.float32),
            jax.ShapeDtypeStruct((N, W), jnp.float32),
        ],
    )(flat, W1m[:, :H].T, W1m[:, H:].T, b1m.reshape(1, H))

    mesh = plsc.VectorSubcoreMesh(core_axis_name="c", subcore_axis_name="s")
    sc_edge = functools.partial(
        pl.kernel,
        out_type=jax.ShapeDtypeStruct((NC, NPAD, W), jnp.float32),
        mesh=mesh,
        compiler_params=pltpu.CompilerParams(use_tc_tiling_on_sc=False),
        scratch_types=[
            pltpu.VMEM((G, C), jnp.int32),
            pltpu.VMEM((G, C), jnp.int32),
            pltpu.VMEM((C, W), jnp.float32),
            pltpu.VMEM((C, W), jnp.float32),
            pltpu.VMEM((C, W), jnp.float32),
            pltpu.VMEM((C, W), jnp.float32),
            pltpu.VMEM((C, W), jnp.float32),
            pltpu.VMEM((C, W), jnp.float32),
            pltpu.VMEM_SHARED((NPAD, W), jnp.float32),
            pltpu.SemaphoreType.DMA,
            pltpu.SemaphoreType.DMA,
            pltpu.SemaphoreType.DMA,
            pltpu.SemaphoreType.DMA,
            pltpu.SemaphoreType.DMA,
            pltpu.SemaphoreType.DMA,
        ],
    )(_sc_edge_body)
    parts = sc_edge(ps, pt, edge_index.reshape(2, E // C, C))

    RE = 1000
    out = pl.pallas_call(
        _epilogue_body,
        grid=(N // RE,),
        in_specs=[
            pl.BlockSpec((NC, RE, W), lambda i: (0, i, 0)),
            pl.BlockSpec((RE, H), lambda i: (i, 0)),
            pl.BlockSpec((H, H), lambda i: (0, 0)),
            pl.BlockSpec((1, H), lambda i: (0, 0)),
            pl.BlockSpec((H, H), lambda i: (0, 0)),
            pl.BlockSpec((H, H), lambda i: (0, 0)),
            pl.BlockSpec((1, H), lambda i: (0, 0)),
            pl.BlockSpec((H, H), lambda i: (0, 0)),
            pl.BlockSpec((1, H), lambda i: (0, 0)),
            pl.BlockSpec((1, H), lambda i: (0, 0)),
            pl.BlockSpec((1, H), lambda i: (0, 0)),
        ],
        out_specs=pl.BlockSpec((RE, H), lambda i: (i, 0)),
        out_shape=jax.ShapeDtypeStruct((N, H), jnp.float32),
    )(parts, flat, W2m.T, b2m.reshape(1, H),
      W1u[:, :H].T, W1u[:, H:].T,
      b1u.reshape(1, H), W2u.T, b2u.reshape(1, H), gamma.reshape(1, H),
      beta.reshape(1, H))

    return out.reshape(1, N, H)


# speedup vs baseline: 2.0097x; 2.0097x over previous
import functools

import jax
import jax.numpy as jnp
from jax import lax
from jax.experimental import pallas as pl
from jax.experimental.pallas import tpu as pltpu
from jax.experimental.pallas import tpu_sc as plsc

N = 10000
H = 128
E = 320000
PAD = 16
W = H + PAD

NC = 2
NS = 16
NPAD = N
RPT = NPAD // NS
EPC = E // NC
EPT = EPC // NS
C = 40
VPR = W // 16
G = 50
NCHUNK = EPT // C
NGROUP = NCHUNK // G


def _prologue_body(x_ref, ws_ref, wt_ref, b1_ref, ps_ref, pt_ref):
    x = x_ref[...]
    ps = jnp.dot(x, ws_ref[...], preferred_element_type=jnp.float32)
    pt = jnp.dot(x, wt_ref[...], preferred_element_type=jnp.float32) + b1_ref[...]
    col = lax.broadcasted_iota(jnp.int32, (x.shape[0], PAD), 1)
    pad = jnp.where(col == 0, 0.5, 0.0).astype(jnp.float32)
    ps_ref[...] = jnp.concatenate([ps, pad], axis=1)
    pt_ref[...] = jnp.concatenate([pt, pad], axis=1)


def _sc_edge_body(ps_hbm, pt_hbm, edge_hbm, out_hbm,
                  sidx, tidx, rs0, rt0, rs1, rt1, w0, w1, acc,
                  sem_s0, sem_t0, sem_s1, sem_t1, sem_w0, sem_w1):
    cid = lax.axis_index("c")
    sid = lax.axis_index("s")

    @pl.loop(0, C)
    def _zero_rows(i):
        for j in range(VPR):
            w0[i, pl.ds(j * 16, 16)] = jnp.zeros((16,), jnp.float32)

    row0 = sid * RPT
    for t in range(15):
        pltpu.sync_copy(w0, acc.at[pl.ds(row0 + t * C, C)])
    pltpu.sync_copy(w0.at[pl.ds(0, 25)], acc.at[pl.ds(row0 + 600, 25)])
    plsc.subcore_barrier()

    rbase = (cid * NS + sid) * NCHUNK

    def start_g(k, rs, rt, sem_s, sem_t):
        pltpu.async_copy(ps_hbm.at[sidx.at[k]], rs, sem_s)
        pltpu.async_copy(pt_hbm.at[tidx.at[k]], rt, sem_t)

    def compute(k, rs, rt, w, sem_s, sem_t):
        pltpu.make_async_copy(ps_hbm.at[sidx.at[k]], rs, sem_s).wait()
        pltpu.make_async_copy(pt_hbm.at[tidx.at[k]], rt, sem_t).wait()

        @pl.loop(0, C)
        def _relu_rows(i):
            for j in range(VPR):
                sl = pl.ds(j * 16, 16)
                w[i, sl] = jnp.maximum(rs[i, sl] + rt[i, sl], 0.0)

    def start_sc(k, w, sem_w):
        pltpu.async_copy(w, acc.at[tidx.at[k]], sem_w, add=True)

    def wait_sc(k, w, sem_w):
        pltpu.make_async_copy(w, acc.at[tidx.at[k]], sem_w).wait()

    @pl.loop(0, NGROUP)
    def _group(g):
        grow = rbase + g * G
        pltpu.sync_copy(edge_hbm.at[0, pl.ds(grow, G)], sidx)
        pltpu.sync_copy(edge_hbm.at[1, pl.ds(grow, G)], tidx)

        start_g(0, rs0, rt0, sem_s0, sem_t0)
        start_g(1, rs1, rt1, sem_s1, sem_t1)
        compute(0, rs0, rt0, w0, sem_s0, sem_t0)
        start_sc(0, w0, sem_w0)
        start_g(2, rs0, rt0, sem_s0, sem_t0)
        compute(1, rs1, rt1, w1, sem_s1, sem_t1)
        start_sc(1, w1, sem_w1)
        start_g(3, rs1, rt1, sem_s1, sem_t1)

        @pl.loop(0, (G - 4) // 2)
        def _pair(m):
            k = 2 * m + 2
            compute(k, rs0, rt0, w0, sem_s0, sem_t0)
            wait_sc(k - 2, w0, sem_w0)
            start_sc(k, w0, sem_w0)
            start_g(k + 2, rs0, rt0, sem_s0, sem_t0)
            compute(k + 1, rs1, rt1, w1, sem_s1, sem_t1)
            wait_sc(k - 1, w1, sem_w1)
            start_sc(k + 1, w1, sem_w1)
            start_g(k + 3, rs1, rt1, sem_s1, sem_t1)

        compute(G - 2, rs0, rt0, w0, sem_s0, sem_t0)
        wait_sc(G - 4, w0, sem_w0)
        start_sc(G - 2, w0, sem_w0)
        compute(G - 1, rs1, rt1, w1, sem_s1, sem_t1)
        wait_sc(G - 3, w1, sem_w1)
        start_sc(G - 1, w1, sem_w1)
        wait_sc(G - 2, w0, sem_w0)
        wait_sc(G - 1, w1, sem_w1)

    plsc.subcore_barrier()
    pltpu.sync_copy(acc.at[pl.ds(row0, RPT)], out_hbm.at[cid, pl.ds(row0, RPT)])


def _epilogue_body(parts_ref, x_ref, w2m_ref, b2m_ref, wus_ref, wua_ref,
                   b1u_ref, w2u_ref, b2u_ref, g_ref, bt_ref, o_ref):
    p = parts_ref[0] + parts_ref[1]
    aggh = p[:, :H]
    cnt = p[:, H:H + 1]
    mh = aggh / jnp.maximum(cnt, 1.0)
    agg = (jnp.dot(mh, w2m_ref[...], preferred_element_type=jnp.float32)
           + b2m_ref[...] * (cnt > 0).astype(jnp.float32))
    x = x_ref[...]
    h2 = jnp.maximum(
        jnp.dot(x, wus_ref[...], preferred_element_type=jnp.float32)
        + jnp.dot(agg, wua_ref[...], preferred_element_type=jnp.float32)
        + b1u_ref[...], 0.0)
    upd = jnp.dot(h2, w2u_ref[...], preferred_element_type=jnp.float32) + b2u_ref[...]
    y = x + upd
    mu = jnp.mean(y, axis=1, keepdims=True)
    var = jnp.mean((y - mu) * (y - mu), axis=1, keepdims=True)
    o_ref[...] = (y - mu) * lax.rsqrt(var + 1e-5) * g_ref[...] + bt_ref[...]


def kernel(node_features, edge_index, W1m, b1m, W2m, b2m,
           W1u, b1u, W2u, b2u, gamma, beta):
    flat = node_features.reshape(N, H)

    RP = 1000
    ps, pt = pl.pallas_call(
        _prologue_body,
        grid=(N // RP,),
        in_specs=[
            pl.BlockSpec((RP, H), lambda i: (i, 0)),
            pl.BlockSpec((H, H), lambda i: (0, 0)),
            pl.BlockSpec((H, H), lambda i: (0, 0)),
            pl.BlockSpec((1, H), lambda i: (0, 0)),
        ],
        out_specs=[
            pl.BlockSpec((RP, W), lambda i: (i, 0)),
            pl.BlockSpec((RP, W), lambda i: (i, 0)),
        ],
        out_shape=[
            jax.ShapeDtypeStruct((N, W), jnp.float32),
            jax.ShapeDtypeStruct((N, W), jnp.float32),
        ],
    )(flat, W1m[:, :H].T, W1m[:, H:].T, b1m.reshape(1, H))

    mesh = plsc.VectorSubcoreMesh(core_axis_name="c", subcore_axis_name="s")
    sc_edge = functools.partial(
        pl.kernel,
        out_type=jax.ShapeDtypeStruct((NC, NPAD, W), jnp.float32),
        mesh=mesh,
        compiler_params=pltpu.CompilerParams(use_tc_tiling_on_sc=False),
        scratch_types=[
            pltpu.VMEM((G, C), jnp.int32),
            pltpu.VMEM((G, C), jnp.int32),
            pltpu.VMEM((C, W), jnp.float32),
            pltpu.VMEM((C, W), jnp.float32),
            pltpu.VMEM((C, W), jnp.float32),
            pltpu.VMEM((C, W), jnp.float32),
            pltpu.VMEM((C, W), jnp.float32),
            pltpu.VMEM((C, W), jnp.float32),
            pltpu.VMEM_SHARED((NPAD, W), jnp.float32),
            pltpu.SemaphoreType.DMA,
            pltpu.SemaphoreType.DMA,
            pltpu.SemaphoreType.DMA,
            pltpu.SemaphoreType.DMA,
            pltpu.SemaphoreType.DMA,
            pltpu.SemaphoreType.DMA,
        ],
    )(_sc_edge_body)
    parts = sc_edge(ps, pt, edge_index.reshape(2, E // C, C))

    RE = 1000
    out = pl.pallas_call(
        _epilogue_body,
        grid=(N // RE,),
        in_specs=[
            pl.BlockSpec((NC, RE, W), lambda i: (0, i, 0)),
            pl.BlockSpec((RE, H), lambda i: (i, 0)),
            pl.BlockSpec((H, H), lambda i: (0, 0)),
            pl.BlockSpec((1, H), lambda i: (0, 0)),
            pl.BlockSpec((H, H), lambda i: (0, 0)),
            pl.BlockSpec((H, H), lambda i: (0, 0)),
            pl.BlockSpec((1, H), lambda i: (0, 0)),
            pl.BlockSpec((H, H), lambda i: (0, 0)),
            pl.BlockSpec((1, H), lambda i: (0, 0)),
            pl.BlockSpec((1, H), lambda i: (0, 0)),
            pl.BlockSpec((1, H), lambda i: (0, 0)),
        ],
        out_specs=pl.BlockSpec((RE, H), lambda i: (i, 0)),
        out_shape=jax.ShapeDtypeStruct((N, H), jnp.float32),
    )(parts, flat, W2m.T, b2m.reshape(1, H),
      W1u[:, :H].T, W1u[:, H:].T,
      b1u.reshape(1, H), W2u.T, b2u.reshape(1, H), gamma.reshape(1, H),
      beta.reshape(1, H))

    return out.reshape(1, N, H)
